# baseline (speedup 1.0000x reference)
import functools

import jax
import jax.numpy as jnp
from jax import lax
from jax.experimental import pallas as pl
from jax.experimental.pallas import tpu as pltpu
from jax.experimental.pallas import tpu_sc as plsc

B, L, D = 4096, 200, 128
N = B * L
G = N // 128
RBm = 320


def _tc_body(mask_ref, w_ref, out_ref):
    m = mask_ref[...]
    w0 = w_ref[0]
    w1 = w_ref[1]
    m3 = jax.lax.broadcast_in_dim(m, (RBm, 128, D), (0, 1))
    out_ref[...] = jnp.where(m3 != 0, w1[None, None, :], w0[None, None, :])


def _tc_kernel(is_target_mask, embedding_weight):
    mask_packed = is_target_mask.astype(jnp.int32).reshape(G, 128)
    grid = (G // RBm,)
    out = pl.pallas_call(
        _tc_body,
        grid=grid,
        in_specs=[
            pl.BlockSpec((RBm, 128), lambda i: (i, 0)),
            pl.BlockSpec((2, D), lambda i: (0, 0)),
        ],
        out_specs=pl.BlockSpec((RBm, 128, D), lambda i: (i, 0, 0)),
        out_shape=jax.ShapeDtypeStruct((G, 128, D), jnp.float32),
        compiler_params=pltpu.CompilerParams(
            dimension_semantics=("parallel",),
        ),
    )(mask_packed, embedding_weight)
    return out.reshape(B, L, D)


NW = 32
ROWS_PW = N // NW
C = 400
NSTEPS = ROWS_PW // C


@functools.partial(
    pl.kernel,
    mesh=plsc.VectorSubcoreMesh(core_axis_name="c", subcore_axis_name="s"),
    out_type=jax.ShapeDtypeStruct((N, D), jnp.float32),
    scratch_types=[
        pltpu.VMEM((ROWS_PW,), jnp.int32),
        pltpu.VMEM((C, D), jnp.float32),
        pltpu.VMEM((C, D), jnp.float32),
        pltpu.VMEM((2, D), jnp.float32),
        pltpu.SemaphoreType.DMA,
        pltpu.SemaphoreType.DMA,
        pltpu.SemaphoreType.DMA,
    ],
)
def _sc_lookup(table_hbm, idx_hbm, out_hbm, idx_all, r0, r1, tab_v, sem_g, so0, so1):
    wid = lax.axis_index("s") * 2 + lax.axis_index("c")
    base = wid * ROWS_PW
    row_bufs = (r0, r1)
    sems_out = (so0, so1)
    pltpu.sync_copy(table_hbm, tab_v)
    pltpu.sync_copy(idx_hbm.at[pl.ds(base, ROWS_PW)], idx_all)

    def fill(b, off):
        buf = row_bufs[b]
        lo = off - base

        def group(g, carry):
            r0_ = g * 16
            mv = idx_all[pl.ds(lo + r0_, 16)]
            for k in range(16):
                m = mv[k]
                r = r0_ + k
                for i in range(8):
                    buf[r, pl.ds(16 * i, 16)] = tab_v[m, pl.ds(16 * i, 16)]
            return carry

        lax.fori_loop(0, C // 16, group, 0)

    def start_store(b, off):
        pltpu.async_copy(row_bufs[b], out_hbm.at[pl.ds(off, C)], sems_out[b])

    def wait_store(b, off):
        pltpu.make_async_copy(
            row_bufs[b], out_hbm.at[pl.ds(off, C)], sems_out[b]
        ).wait()

    for b in (0, 1):
        fill(b, base + b * C)
        start_store(b, base + b * C)

    def step(jj, carry):
        off2 = base + jj * 2 * C
        for b in (0, 1):
            off = off2 + b * C
            wait_store(b, off - 2 * C)
            fill(b, off)
            start_store(b, off)
        return carry

    lax.fori_loop(1, NSTEPS // 2, step, 0)
    for b in (0, 1):
        wait_store(b, base + (NSTEPS - 2 + b) * C)


def _sc_kernel(is_target_mask, embedding_weight):
    idx = is_target_mask.astype(jnp.int32).reshape(N)
    out = _sc_lookup(embedding_weight, idx)
    return out.reshape(B, L, D)


def kernel(is_target_mask, embedding_weight):
    return _sc_kernel(is_target_mask, embedding_weight)


# speedup vs baseline: 1.8156x; 1.8156x over previous
import functools

import jax
import jax.numpy as jnp
from jax import lax
from jax.experimental import pallas as pl
from jax.experimental.pallas import tpu as pltpu
from jax.experimental.pallas import tpu_sc as plsc

B, L, D = 4096, 200, 128
N = B * L

N_SC = 256000
N_TC = N - N_SC
G_TC = N_TC // 128
RBm = 200


def _tc_body(mask_ref, w_ref, out_ref):
    m = mask_ref[...]
    w0 = w_ref[0]
    w1 = w_ref[1]
    m3 = jax.lax.broadcast_in_dim(m, (RBm, 128, D), (0, 1))
    out_ref[...] = jnp.where(m3 != 0, w1[None, None, :], w0[None, None, :])


def _tc_part(mask_packed, embedding_weight):
    grid = (G_TC // RBm,)
    out = pl.pallas_call(
        _tc_body,
        grid=grid,
        in_specs=[
            pl.BlockSpec((RBm, 128), lambda i: (i, 0)),
            pl.BlockSpec((2, D), lambda i: (0, 0)),
        ],
        out_specs=pl.BlockSpec((RBm, 128, D), lambda i: (i, 0, 0)),
        out_shape=jax.ShapeDtypeStruct((G_TC, 128, D), jnp.float32),
        compiler_params=pltpu.CompilerParams(
            dimension_semantics=("parallel",),
        ),
    )(mask_packed, embedding_weight)
    return out.reshape(N_TC, D)


NW = 32
ROWS_PW = N_SC // NW
C = 400
NSTEPS = ROWS_PW // C


@functools.partial(
    pl.kernel,
    mesh=plsc.VectorSubcoreMesh(core_axis_name="c", subcore_axis_name="s"),
    out_type=jax.ShapeDtypeStruct((N_SC, D), jnp.float32),
    scratch_types=[
        pltpu.VMEM((ROWS_PW,), jnp.int32),
        pltpu.VMEM((C, D), jnp.float32),
        pltpu.VMEM((C, D), jnp.float32),
        pltpu.VMEM_SHARED((2, D), jnp.float32),
        pltpu.SemaphoreType.DMA,
        pltpu.SemaphoreType.DMA,
        pltpu.SemaphoreType.DMA,
    ],
)
def _sc_lookup(table_hbm, idx_hbm, out_hbm, idx_all, r0, r1, tab_v, sem_g, so0, so1):
    wid = lax.axis_index("s") * 2 + lax.axis_index("c")
    base = wid * ROWS_PW
    row_bufs = (r0, r1)
    sems_out = (so0, so1)
    pltpu.sync_copy(table_hbm, tab_v)
    pltpu.sync_copy(idx_hbm.at[pl.ds(base, ROWS_PW)], idx_all)

    def fill(b, off):
        pltpu.async_copy(
            tab_v.at[idx_all.at[pl.ds(off - base, C)]], row_bufs[b], sem_g
        ).wait()

    def start_store(b, off):
        pltpu.async_copy(row_bufs[b], out_hbm.at[pl.ds(off, C)], sems_out[b])

    def wait_store(b, off):
        pltpu.make_async_copy(
            row_bufs[b], out_hbm.at[pl.ds(off, C)], sems_out[b]
        ).wait()

    for b in (0, 1):
        fill(b, base + b * C)
        start_store(b, base + b * C)

    def step(jj, carry):
        off2 = base + jj * 2 * C
        for b in (0, 1):
            off = off2 + b * C
            wait_store(b, off - 2 * C)
            fill(b, off)
            start_store(b, off)
        return carry

    lax.fori_loop(1, NSTEPS // 2, step, 0)
    for b in (0, 1):
        wait_store(b, base + (NSTEPS - 2 + b) * C)


def kernel(is_target_mask, embedding_weight):
    idx = is_target_mask.astype(jnp.int32).reshape(N)
    mask_packed_tc = idx[:N_TC].reshape(G_TC, 128)
    out_tc = _tc_part(mask_packed_tc, embedding_weight)
    out_sc = _sc_lookup(embedding_weight, idx[N_TC:])
    out = jnp.concatenate([out_tc, out_sc], axis=0)
    return out.reshape(B, L, D)


# speedup vs baseline: 2.7860x; 1.5345x over previous
import functools

import jax
import jax.numpy as jnp
from jax import lax
from jax.experimental import pallas as pl
from jax.experimental.pallas import tpu as pltpu
from jax.experimental.pallas import tpu_sc as plsc

B, L, D = 4096, 200, 128
N = B * L

NW = 32
ROWS_PW = N // NW
C = 400
NSTEPS = ROWS_PW // C


@functools.partial(
    pl.kernel,
    mesh=plsc.VectorSubcoreMesh(core_axis_name="c", subcore_axis_name="s"),
    out_type=jax.ShapeDtypeStruct((N, D), jnp.float32),
    scratch_types=[
        pltpu.VMEM((ROWS_PW,), jnp.int32),
        pltpu.VMEM((C, D), jnp.float32),
        pltpu.VMEM((C, D), jnp.float32),
        pltpu.VMEM_SHARED((2, D), jnp.float32),
        pltpu.SemaphoreType.DMA,
        pltpu.SemaphoreType.DMA,
        pltpu.SemaphoreType.DMA,
    ],
)
def _sc_lookup(table_hbm, idx_hbm, out_hbm, idx_all, r0, r1, tab_v, sem_g, so0, so1):
    wid = lax.axis_index("s") * 2 + lax.axis_index("c")
    base = wid * ROWS_PW
    row_bufs = (r0, r1)
    sems_out = (so0, so1)
    pltpu.sync_copy(table_hbm, tab_v)
    pltpu.sync_copy(idx_hbm.at[pl.ds(base, ROWS_PW)], idx_all)

    def fill(b, off):
        pltpu.async_copy(
            tab_v.at[idx_all.at[pl.ds(off - base, C)]], row_bufs[b], sem_g
        ).wait()

    def start_store(b, off):
        pltpu.async_copy(row_bufs[b], out_hbm.at[pl.ds(off, C)], sems_out[b])

    def wait_store(b, off):
        pltpu.make_async_copy(
            row_bufs[b], out_hbm.at[pl.ds(off, C)], sems_out[b]
        ).wait()

    for b in (0, 1):
        fill(b, base + b * C)
        start_store(b, base + b * C)

    def step(jj, carry):
        off2 = base + jj * 2 * C
        for b in (0, 1):
            off = off2 + b * C
            wait_store(b, off - 2 * C)
            fill(b, off)
            start_store(b, off)
        return carry

    lax.fori_loop(1, NSTEPS // 2, step, 0)
    for b in (0, 1):
        wait_store(b, base + (NSTEPS - 2 + b) * C)


def kernel(is_target_mask, embedding_weight):
    idx = is_target_mask.astype(jnp.int32).reshape(N)
    out = _sc_lookup(embedding_weight, idx)
    return out.reshape(B, L, D)


# speedup vs baseline: 2.8043x; 1.0066x over previous
import functools

import jax
import jax.numpy as jnp
from jax import lax
from jax.experimental import pallas as pl
from jax.experimental.pallas import tpu as pltpu
from jax.experimental.pallas import tpu_sc as plsc

B, L, D = 4096, 200, 128
N = B * L

NW = 32
ROWS_PW = N // NW
C = 320
NSTEPS = ROWS_PW // C


@functools.partial(
    pl.kernel,
    mesh=plsc.VectorSubcoreMesh(core_axis_name="c", subcore_axis_name="s"),
    out_type=jax.ShapeDtypeStruct((N, D), jnp.float32),
    scratch_types=[
        pltpu.VMEM((ROWS_PW,), jnp.int32),
        pltpu.VMEM((C, D), jnp.float32),
        pltpu.VMEM((C, D), jnp.float32),
        pltpu.VMEM_SHARED((2, D), jnp.float32),
        pltpu.SemaphoreType.DMA,
        pltpu.SemaphoreType.DMA,
        pltpu.SemaphoreType.DMA,
    ],
)
def _sc_lookup(table_hbm, idx_hbm, out_hbm, idx_all, r0, r1, tab_v, sem_g, so0, so1):
    wid = lax.axis_index("s") * 2 + lax.axis_index("c")
    base = wid * ROWS_PW
    row_bufs = (r0, r1)
    sems_out = (so0, so1)
    pltpu.sync_copy(table_hbm, tab_v)
    pltpu.sync_copy(idx_hbm.at[pl.ds(base, ROWS_PW)], idx_all)

    def fill(b, off):
        pltpu.async_copy(
            tab_v.at[idx_all.at[pl.ds(off - base, C)]], row_bufs[b], sem_g
        ).wait()

    def start_store(b, off):
        pltpu.async_copy(row_bufs[b], out_hbm.at[pl.ds(off, C)], sems_out[b])

    def wait_store(b, off):
        pltpu.make_async_copy(
            row_bufs[b], out_hbm.at[pl.ds(off, C)], sems_out[b]
        ).wait()

    for b in (0, 1):
        fill(b, base + b * C)
        start_store(b, base + b * C)

    def step(jj, carry):
        off2 = base + jj * 2 * C
        for b in (0, 1):
            off = off2 + b * C
            wait_store(b, off - 2 * C)
            fill(b, off)
            start_store(b, off)
        return carry

    lax.fori_loop(1, NSTEPS // 2, step, 0)
    for b in (0, 1):
        wait_store(b, base + (NSTEPS - 2 + b) * C)


def kernel(is_target_mask, embedding_weight):
    idx = is_target_mask.astype(jnp.int32).reshape(N)
    out = _sc_lookup(embedding_weight, idx)
    return out.reshape(B, L, D)


# speedup vs baseline: 2.9120x; 1.0384x over previous
import functools

import jax
import jax.numpy as jnp
from jax import lax
from jax.experimental import pallas as pl
from jax.experimental.pallas import tpu as pltpu
from jax.experimental.pallas import tpu_sc as plsc

B, L, D = 4096, 200, 128
N = B * L

NW = 32
ROWS_PW = N // NW
C = 200
NSTEPS = ROWS_PW // C


@functools.partial(
    pl.kernel,
    mesh=plsc.VectorSubcoreMesh(core_axis_name="c", subcore_axis_name="s"),
    out_type=jax.ShapeDtypeStruct((N, D), jnp.float32),
    scratch_types=[
        pltpu.VMEM((ROWS_PW,), jnp.int32),
        pltpu.VMEM((C, D), jnp.float32),
        pltpu.VMEM((C, D), jnp.float32),
        pltpu.VMEM_SHARED((2, D), jnp.float32),
        pltpu.SemaphoreType.DMA,
        pltpu.SemaphoreType.DMA,
        pltpu.SemaphoreType.DMA,
    ],
)
def _sc_lookup(table_hbm, idx_hbm, out_hbm, idx_all, r0, r1, tab_v, sem_g, so0, so1):
    wid = lax.axis_index("s") * 2 + lax.axis_index("c")
    base = wid * ROWS_PW
    row_bufs = (r0, r1)
    sems_out = (so0, so1)
    pltpu.sync_copy(table_hbm, tab_v)
    pltpu.sync_copy(idx_hbm.at[pl.ds(base, ROWS_PW)], idx_all)

    def fill(b, off):
        pltpu.async_copy(
            tab_v.at[idx_all.at[pl.ds(off - base, C)]], row_bufs[b], sem_g
        ).wait()

    def start_store(b, off):
        pltpu.async_copy(row_bufs[b], out_hbm.at[pl.ds(off, C)], sems_out[b])

    def wait_store(b, off):
        pltpu.make_async_copy(
            row_bufs[b], out_hbm.at[pl.ds(off, C)], sems_out[b]
        ).wait()

    for b in (0, 1):
        fill(b, base + b * C)
        start_store(b, base + b * C)

    def step(jj, carry):
        off2 = base + jj * 2 * C
        for b in (0, 1):
            off = off2 + b * C
            wait_store(b, off - 2 * C)
            fill(b, off)
            start_store(b, off)
        return carry

    lax.fori_loop(1, NSTEPS // 2, step, 0)
    for b in (0, 1):
        wait_store(b, base + (NSTEPS - 2 + b) * C)


def kernel(is_target_mask, embedding_weight):
    idx = is_target_mask.astype(jnp.int32).reshape(N)
    out = _sc_lookup(embedding_weight, idx)
    return out.reshape(B, L, D)


# speedup vs baseline: 3.0432x; 1.0451x over previous
import functools

import jax
import jax.numpy as jnp
from jax import lax
from jax.experimental import pallas as pl
from jax.experimental.pallas import tpu as pltpu
from jax.experimental.pallas import tpu_sc as plsc

B, L, D = 4096, 200, 128
N = B * L

NW = 32
ROWS_PW = N // NW
C = 128
NSTEPS = ROWS_PW // C


@functools.partial(
    pl.kernel,
    mesh=plsc.VectorSubcoreMesh(core_axis_name="c", subcore_axis_name="s"),
    out_type=jax.ShapeDtypeStruct((N, D), jnp.float32),
    scratch_types=[
        pltpu.VMEM((ROWS_PW,), jnp.int32),
        pltpu.VMEM((C, D), jnp.float32),
        pltpu.VMEM((C, D), jnp.float32),
        pltpu.VMEM_SHARED((2, D), jnp.float32),
        pltpu.SemaphoreType.DMA,
        pltpu.SemaphoreType.DMA,
        pltpu.SemaphoreType.DMA,
    ],
)
def _sc_lookup(table_hbm, idx_hbm, out_hbm, idx_all, r0, r1, tab_v, sem_g, so0, so1):
    wid = lax.axis_index("s") * 2 + lax.axis_index("c")
    base = wid * ROWS_PW
    row_bufs = (r0, r1)
    sems_out = (so0, so1)
    pltpu.sync_copy(table_hbm, tab_v)
    pltpu.sync_copy(idx_hbm.at[pl.ds(base, ROWS_PW)], idx_all)

    def fill(b, off):
        pltpu.async_copy(
            tab_v.at[idx_all.at[pl.ds(off - base, C)]], row_bufs[b], sem_g
        ).wait()

    def start_store(b, off):
        pltpu.async_copy(row_bufs[b], out_hbm.at[pl.ds(off, C)], sems_out[b])

    def wait_store(b, off):
        pltpu.make_async_copy(
            row_bufs[b], out_hbm.at[pl.ds(off, C)], sems_out[b]
        ).wait()

    for b in (0, 1):
        fill(b, base + b * C)
        start_store(b, base + b * C)

    def step(jj, carry):
        off2 = base + jj * 2 * C
        for b in (0, 1):
            off = off2 + b * C
            wait_store(b, off - 2 * C)
            fill(b, off)
            start_store(b, off)
        return carry

    lax.fori_loop(1, NSTEPS // 2, step, 0)
    for b in (0, 1):
        wait_store(b, base + (NSTEPS - 2 + b) * C)


def kernel(is_target_mask, embedding_weight):
    idx = is_target_mask.astype(jnp.int32).reshape(N)
    out = _sc_lookup(embedding_weight, idx)
    return out.reshape(B, L, D)
